# baseline (speedup 1.0000x reference)
import functools

import jax
import jax.numpy as jnp
from jax import lax
from jax.experimental import pallas as pl
from jax.experimental.pallas import tpu as pltpu
from jax.experimental.pallas import tpu_sc as plsc

NC = 2
NS = 16
NW = NC * NS
EBLK = 64
NBUF = 3
CHUNKS = tuple(range(0, EBLK - 15, 16)) + (() if EBLK % 16 == 0 else (EBLK - 16,))
PADR = 16


def _zero_vmem_1d(ref, n):
  zeros = jnp.zeros((16,), ref.dtype)

  def body(i):
    ref[pl.ds(i * 16, 16)] = zeros

  pl.loop(0, n // 16)(body)


def _make_deg_kernel(e, n):
  epw = e // NW
  nblk = epw // EBLK
  mesh = plsc.VectorSubcoreMesh(
      core_axis_name="c", subcore_axis_name="s", num_cores=NC, num_subcores=NS)

  nbuf = NBUF
  assert nblk % nbuf == 0

  @functools.partial(
      pl.kernel,
      mesh=mesh,
      out_type=jax.ShapeDtypeStruct((NC, n + PADR), jnp.float32),
      scratch_types=[
          pltpu.VMEM_SHARED((n + PADR,), jnp.float32),
          pltpu.VMEM((n + PADR,), jnp.float32),
          pltpu.VMEM((EBLK,), jnp.float32),
          pltpu.VMEM((epw,), jnp.int32),
      ]
      + [pltpu.VMEM((EBLK,), jnp.int32) for _ in range(nbuf)]
      + [pltpu.SemaphoreType.DMA for _ in range(nbuf + 1)],
  )
  def deg_kernel(dst_hbm, out_hbm, deg_sh, zbuf, ones_v, didx, *rest):
    dbufs = rest[:nbuf]
    ssems = rest[nbuf:2 * nbuf]
    isem = rest[2 * nbuf]
    cid = lax.axis_index("c")
    sid = lax.axis_index("s")
    wid = sid * NC + cid

    pltpu.async_copy(dst_hbm.at[wid], didx, isem)

    ones16 = jnp.ones((16,), jnp.float32)
    for o in CHUNKS:
      ones_v[pl.ds(o, 16)] = ones16

    @pl.when(sid == 0)
    def _():
      _zero_vmem_1d(zbuf, n + PADR)
      pltpu.sync_copy(zbuf, deg_sh)

    pltpu.make_async_copy(dst_hbm.at[wid], didx, isem).wait()
    plsc.subcore_barrier()

    def wait_scatter(k):
      pltpu.make_async_copy(ones_v, deg_sh.at[dbufs[k]], ssems[k]).wait()

    def body(i):
      for k in range(nbuf):
        j = i * nbuf + k

        @pl.when(i >= 1)
        def _():
          wait_scatter(k)

        base = j * EBLK
        for o in CHUNKS:
          dbufs[k][pl.ds(o, 16)] = didx[pl.ds(base + o, 16)]
        pltpu.async_copy(ones_v, deg_sh.at[dbufs[k]], ssems[k], add=True)

    pl.loop(0, nblk // nbuf)(body)

    for k in range(nbuf):
      wait_scatter(k)

    plsc.subcore_barrier()

    @pl.when(sid == 0)
    def _():
      pltpu.sync_copy(deg_sh, out_hbm.at[cid])

  return deg_kernel


def _make_scatter_kernel(e, n, d):
  epw = e // NW
  nblk = epw // EBLK
  wtiles = 10
  rpt = n // wtiles
  mesh = plsc.VectorSubcoreMesh(
      core_axis_name="c", subcore_axis_name="s", num_cores=NC, num_subcores=NS)

  nbuf = NBUF
  zr = 40
  assert nblk % nbuf == 0 and nblk > 2 * nbuf and rpt % zr == 0 and EBLK >= zr

  @functools.partial(
      pl.kernel,
      mesh=mesh,
      out_type=jax.ShapeDtypeStruct((NC, n, d), jnp.float32),
      scratch_types=[
          pltpu.VMEM_SHARED((n + PADR, d), jnp.float32),
          pltpu.VMEM((epw,), jnp.int32),
          pltpu.VMEM((epw,), jnp.int32),
      ]
      + [pltpu.VMEM((EBLK, d), jnp.float32) for _ in range(nbuf)]
      + [pltpu.VMEM((EBLK,), jnp.int32) for _ in range(nbuf)]
      + [pltpu.SemaphoreType.DMA for _ in range(2 * nbuf)],
  )
  def scatter_kernel(hp_hbm, src_hbm, dst_hbm, out_hbm,
                     acc_sh, sidx, didx, *bufs_and_sems):
    bufs = bufs_and_sems[:nbuf]
    dbufs = bufs_and_sems[nbuf:2 * nbuf]
    gsems = bufs_and_sems[2 * nbuf:3 * nbuf]
    ssems = bufs_and_sems[3 * nbuf:]
    cid = lax.axis_index("c")
    sid = lax.axis_index("s")
    wid = sid * NC + cid

    pltpu.async_copy(src_hbm.at[wid], sidx, gsems[0])
    pltpu.async_copy(dst_hbm.at[wid], didx, gsems[1])

    zeros16 = jnp.zeros((16,), jnp.float32)

    def zfill(i):
      def zrow(c):
        bufs[0][i, pl.ds(c * 16, 16)] = zeros16
      pl.loop(0, d // 16)(zrow)

    pl.loop(0, EBLK)(zfill)

    @pl.when(sid < wtiles)
    def _():
      def zcopy(i):
        pltpu.sync_copy(bufs[0].at[pl.ds(0, zr)],
                        acc_sh.at[pl.ds(sid * rpt + i * zr, zr)])

      pl.loop(0, rpt // zr)(zcopy)

    pltpu.make_async_copy(src_hbm.at[wid], sidx, gsems[0]).wait()
    pltpu.make_async_copy(dst_hbm.at[wid], didx, gsems[1]).wait()
    plsc.subcore_barrier()

    def issue_block(j, k):
      pltpu.async_copy(hp_hbm.at[sidx.at[pl.ds(j * EBLK, EBLK)]],
                       bufs[k], gsems[k])
      base = j * EBLK
      for o in CHUNKS:
        dbufs[k][pl.ds(o, 16)] = didx[pl.ds(base + o, 16)]

    def wait_block(k):
      pltpu.make_async_copy(hp_hbm.at[sidx.at[pl.ds(0, EBLK)]],
                            bufs[k], gsems[k]).wait()

    def issue_scatter(k):
      pltpu.async_copy(bufs[k], acc_sh.at[dbufs[k]], ssems[k], add=True)

    def wait_scatter(k):
      pltpu.make_async_copy(bufs[k], acc_sh.at[dbufs[k]], ssems[k]).wait()

    for b in range(nbuf - 1):
      issue_block(b, b)

    def body(i):
      for k in range(nbuf):
        j = i * nbuf + k
        kn = (k + nbuf - 1) % nbuf
        wait_block(k)
        issue_scatter(k)

        @pl.when(j + nbuf - 1 < nblk)
        def _():
          @pl.when(j >= 1)
          def _():
            wait_scatter(kn)

          issue_block(j + nbuf - 1, kn)

    pl.loop(0, nblk // nbuf)(body)

    for k in range(nbuf):
      wait_scatter(k)

    plsc.subcore_barrier()

    @pl.when(sid < wtiles)
    def _():
      pltpu.sync_copy(acc_sh.at[pl.ds(sid * rpt, rpt)],
                      out_hbm.at[cid, pl.ds(sid * rpt, rpt)])

  return scatter_kernel


def _matscale_body(x_ref, w_ref, d_ref, o_ref):
  o_ref[...] = (
      jnp.dot(x_ref[...], w_ref[...], preferred_element_type=jnp.float32)
      * d_ref[...]
  )


def _matscale(x, w, d2, rblk):
  n, dim = x.shape
  grid = n // rblk
  return pl.pallas_call(
      _matscale_body,
      grid=(grid,),
      in_specs=[
          pl.BlockSpec((rblk, dim), lambda i: (i, 0)),
          pl.BlockSpec((dim, dim), lambda i: (0, 0)),
          pl.BlockSpec((rblk, 1), lambda i: (i, 0)),
      ],
      out_specs=pl.BlockSpec((rblk, dim), lambda i: (i, 0)),
      out_shape=jax.ShapeDtypeStruct((n, dim), jnp.float32),
  )(x, w, d2)


def _step_body(p0_ref, p1_ref, hp_ref, d_ref, b_ref, w_ref, hpn_ref, s_ref):
  xv = d_ref[...] * (p0_ref[...] + p1_ref[...] + hp_ref[...]) + b_ref[...]

  @pl.when(pl.program_id(0) == 0)
  def _():
    s_ref[...] = jnp.zeros_like(s_ref)

  s_ref[...] += jnp.sum(xv, axis=0, keepdims=True)
  hpn_ref[...] = (
      jnp.dot(xv, w_ref[...], preferred_element_type=jnp.float32) * d_ref[...]
  )


def _step(p0, p1, hp, d2, b2, w, rblk):
  n, dim = hp.shape
  grid = n // rblk
  return pl.pallas_call(
      _step_body,
      grid=(grid,),
      in_specs=[
          pl.BlockSpec((rblk, dim), lambda i: (i, 0)),
          pl.BlockSpec((rblk, dim), lambda i: (i, 0)),
          pl.BlockSpec((rblk, dim), lambda i: (i, 0)),
          pl.BlockSpec((rblk, 1), lambda i: (i, 0)),
          pl.BlockSpec((1, dim), lambda i: (0, 0)),
          pl.BlockSpec((dim, dim), lambda i: (0, 0)),
      ],
      out_specs=[
          pl.BlockSpec((rblk, dim), lambda i: (i, 0)),
          pl.BlockSpec((1, dim), lambda i: (0, 0)),
      ],
      out_shape=[
          jax.ShapeDtypeStruct((n, dim), jnp.float32),
          jax.ShapeDtypeStruct((1, dim), jnp.float32),
      ],
  )(p0, p1, hp, d2, b2, w)


def _colsum_body(p0_ref, p1_ref, hp_ref, d_ref, b_ref, s_ref):
  xv = d_ref[...] * (p0_ref[...] + p1_ref[...] + hp_ref[...]) + b_ref[...]

  @pl.when(pl.program_id(0) == 0)
  def _():
    s_ref[...] = jnp.zeros_like(s_ref)

  s_ref[...] += jnp.sum(xv, axis=0, keepdims=True)


def _colsum(p0, p1, hp, d2, b2, rblk):
  n, dim = hp.shape
  grid = n // rblk
  return pl.pallas_call(
      _colsum_body,
      grid=(grid,),
      in_specs=[
          pl.BlockSpec((rblk, dim), lambda i: (i, 0)),
          pl.BlockSpec((rblk, dim), lambda i: (i, 0)),
          pl.BlockSpec((rblk, dim), lambda i: (i, 0)),
          pl.BlockSpec((rblk, 1), lambda i: (i, 0)),
          pl.BlockSpec((1, dim), lambda i: (0, 0)),
      ],
      out_specs=pl.BlockSpec((1, dim), lambda i: (0, 0)),
      out_shape=jax.ShapeDtypeStruct((1, dim), jnp.float32),
  )(p0, p1, hp, d2, b2)


def kernel(x, edge_index, W1, b1, W2, b2, W3, b3, W4, b4):
  n, dim = x.shape
  e = edge_index.shape[1]
  assert dim % 16 == 0

  grain = NW * EBLK * NBUF
  ep = ((e + grain - 1) // grain) * grain
  npad = ep - e
  epw = ep // NW
  nblk = epw // EBLK
  src_flat = edge_index[0]
  dst_flat = edge_index[1]
  if npad:
    ar = jnp.arange(npad, dtype=jnp.int32)
    src_flat = jnp.concatenate([src_flat, (ar * 197) % n])
    dst_flat = jnp.concatenate([dst_flat, n + (ar % PADR)])
  src2 = src_flat.reshape(NW, epw)
  dst2 = dst_flat.reshape(NW, epw)

  deg_parts = _make_deg_kernel(ep, n)(dst2)
  deg = deg_parts[0, :n] + deg_parts[1, :n] + 1.0
  d2 = jnp.where(deg > 0, lax.rsqrt(deg), 0.0)[:, None]

  scatter = _make_scatter_kernel(ep, n, dim)
  rblk = 1000

  sums = []
  hp = _matscale(x, W1, d2, rblk)
  for b, wn in ((b1, W2), (b2, W3), (b3, W4)):
    parts = scatter(hp, src2, dst2)
    hp, s = _step(parts[0], parts[1], hp, d2, b.reshape(1, dim), wn, rblk)
    sums.append(s[0])
  parts = scatter(hp, src2, dst2)
  s = _colsum(parts[0], parts[1], hp, d2, b4.reshape(1, dim), rblk)
  sums.append(s[0])

  return jnp.concatenate(sums) / n


# speedup vs baseline: 1.0266x; 1.0266x over previous
import functools

import jax
import jax.numpy as jnp
from jax import lax
from jax.experimental import pallas as pl
from jax.experimental.pallas import tpu as pltpu
from jax.experimental.pallas import tpu_sc as plsc

NC = 2
NS = 16
NW = NC * NS
EBLK = 40
NBUF = 4
CHUNKS = tuple(range(0, EBLK - 15, 16)) + (() if EBLK % 16 == 0 else (EBLK - 16,))
PADR = 16


def _zero_vmem_1d(ref, n):
  zeros = jnp.zeros((16,), ref.dtype)

  def body(i):
    ref[pl.ds(i * 16, 16)] = zeros

  pl.loop(0, n // 16)(body)


def _make_deg_kernel(e, n):
  epw = e // NW
  nblk = epw // EBLK
  mesh = plsc.VectorSubcoreMesh(
      core_axis_name="c", subcore_axis_name="s", num_cores=NC, num_subcores=NS)

  nbuf = NBUF
  assert nblk % nbuf == 0

  @functools.partial(
      pl.kernel,
      mesh=mesh,
      out_type=jax.ShapeDtypeStruct((NC, n + PADR), jnp.float32),
      scratch_types=[
          pltpu.VMEM_SHARED((n + PADR,), jnp.float32),
          pltpu.VMEM((n + PADR,), jnp.float32),
          pltpu.VMEM((EBLK,), jnp.float32),
          pltpu.VMEM((epw,), jnp.int32),
      ]
      + [pltpu.VMEM((EBLK,), jnp.int32) for _ in range(nbuf)]
      + [pltpu.SemaphoreType.DMA for _ in range(nbuf + 1)],
  )
  def deg_kernel(dst_hbm, out_hbm, deg_sh, zbuf, ones_v, didx, *rest):
    dbufs = rest[:nbuf]
    ssems = rest[nbuf:2 * nbuf]
    isem = rest[2 * nbuf]
    cid = lax.axis_index("c")
    sid = lax.axis_index("s")
    wid = sid * NC + cid

    pltpu.async_copy(dst_hbm.at[wid], didx, isem)

    ones16 = jnp.ones((16,), jnp.float32)
    for o in CHUNKS:
      ones_v[pl.ds(o, 16)] = ones16

    @pl.when(sid == 0)
    def _():
      _zero_vmem_1d(zbuf, n + PADR)
      pltpu.sync_copy(zbuf, deg_sh)

    pltpu.make_async_copy(dst_hbm.at[wid], didx, isem).wait()
    plsc.subcore_barrier()

    def wait_scatter(k):
      pltpu.make_async_copy(ones_v, deg_sh.at[dbufs[k]], ssems[k]).wait()

    def body(i):
      for k in range(nbuf):
        j = i * nbuf + k

        @pl.when(i >= 1)
        def _():
          wait_scatter(k)

        base = j * EBLK
        for o in CHUNKS:
          dbufs[k][pl.ds(o, 16)] = didx[pl.ds(base + o, 16)]
        pltpu.async_copy(ones_v, deg_sh.at[dbufs[k]], ssems[k], add=True)

    pl.loop(0, nblk // nbuf)(body)

    for k in range(nbuf):
      wait_scatter(k)

    plsc.subcore_barrier()

    @pl.when(sid == 0)
    def _():
      pltpu.sync_copy(deg_sh, out_hbm.at[cid])

  return deg_kernel


def _make_scatter_kernel(e, n, d):
  epw = e // NW
  nblk = epw // EBLK
  wtiles = 10
  rpt = n // wtiles
  mesh = plsc.VectorSubcoreMesh(
      core_axis_name="c", subcore_axis_name="s", num_cores=NC, num_subcores=NS)

  nbuf = NBUF
  zr = 40
  assert nblk % nbuf == 0 and nblk > 2 * nbuf and rpt % zr == 0 and EBLK >= zr

  @functools.partial(
      pl.kernel,
      mesh=mesh,
      out_type=jax.ShapeDtypeStruct((NC, n, d), jnp.float32),
      scratch_types=[
          pltpu.VMEM_SHARED((n + PADR, d), jnp.float32),
          pltpu.VMEM((epw,), jnp.int32),
          pltpu.VMEM((epw,), jnp.int32),
      ]
      + [pltpu.VMEM((EBLK, d), jnp.float32) for _ in range(nbuf)]
      + [pltpu.VMEM((EBLK,), jnp.int32) for _ in range(nbuf)]
      + [pltpu.SemaphoreType.DMA for _ in range(2 * nbuf)],
  )
  def scatter_kernel(hp_hbm, src_hbm, dst_hbm, out_hbm,
                     acc_sh, sidx, didx, *bufs_and_sems):
    bufs = bufs_and_sems[:nbuf]
    dbufs = bufs_and_sems[nbuf:2 * nbuf]
    gsems = bufs_and_sems[2 * nbuf:3 * nbuf]
    ssems = bufs_and_sems[3 * nbuf:]
    cid = lax.axis_index("c")
    sid = lax.axis_index("s")
    wid = sid * NC + cid

    pltpu.async_copy(src_hbm.at[wid], sidx, gsems[0])
    pltpu.async_copy(dst_hbm.at[wid], didx, gsems[1])

    zeros16 = jnp.zeros((16,), jnp.float32)

    def zfill(i):
      def zrow(c):
        bufs[0][i, pl.ds(c * 16, 16)] = zeros16
      pl.loop(0, d // 16)(zrow)

    pl.loop(0, EBLK)(zfill)

    @pl.when(sid < wtiles)
    def _():
      def zcopy(i):
        pltpu.sync_copy(bufs[0].at[pl.ds(0, zr)],
                        acc_sh.at[pl.ds(sid * rpt + i * zr, zr)])

      pl.loop(0, rpt // zr)(zcopy)

    pltpu.make_async_copy(src_hbm.at[wid], sidx, gsems[0]).wait()
    pltpu.make_async_copy(dst_hbm.at[wid], didx, gsems[1]).wait()
    plsc.subcore_barrier()

    def issue_block(j, k):
      pltpu.async_copy(hp_hbm.at[sidx.at[pl.ds(j * EBLK, EBLK)]],
                       bufs[k], gsems[k])
      base = j * EBLK
      for o in CHUNKS:
        dbufs[k][pl.ds(o, 16)] = didx[pl.ds(base + o, 16)]

    def wait_block(k):
      pltpu.make_async_copy(hp_hbm.at[sidx.at[pl.ds(0, EBLK)]],
                            bufs[k], gsems[k]).wait()

    def issue_scatter(k):
      pltpu.async_copy(bufs[k], acc_sh.at[dbufs[k]], ssems[k], add=True)

    def wait_scatter(k):
      pltpu.make_async_copy(bufs[k], acc_sh.at[dbufs[k]], ssems[k]).wait()

    for b in range(nbuf - 1):
      issue_block(b, b)

    def body(i):
      for k in range(nbuf):
        j = i * nbuf + k
        kn = (k + nbuf - 1) % nbuf
        wait_block(k)
        issue_scatter(k)

        @pl.when(j + nbuf - 1 < nblk)
        def _():
          @pl.when(j >= 1)
          def _():
            wait_scatter(kn)

          issue_block(j + nbuf - 1, kn)

    pl.loop(0, nblk // nbuf)(body)

    for k in range(nbuf):
      wait_scatter(k)

    plsc.subcore_barrier()

    @pl.when(sid < wtiles)
    def _():
      pltpu.sync_copy(acc_sh.at[pl.ds(sid * rpt, rpt)],
                      out_hbm.at[cid, pl.ds(sid * rpt, rpt)])

  return scatter_kernel


def _matscale_body(x_ref, w_ref, d_ref, o_ref):
  o_ref[...] = (
      jnp.dot(x_ref[...], w_ref[...], preferred_element_type=jnp.float32)
      * d_ref[...]
  )


def _matscale(x, w, d2, rblk):
  n, dim = x.shape
  grid = n // rblk
  return pl.pallas_call(
      _matscale_body,
      grid=(grid,),
      in_specs=[
          pl.BlockSpec((rblk, dim), lambda i: (i, 0)),
          pl.BlockSpec((dim, dim), lambda i: (0, 0)),
          pl.BlockSpec((rblk, 1), lambda i: (i, 0)),
      ],
      out_specs=pl.BlockSpec((rblk, dim), lambda i: (i, 0)),
      out_shape=jax.ShapeDtypeStruct((n, dim), jnp.float32),
  )(x, w, d2)


def _step_body(p0_ref, p1_ref, hp_ref, d_ref, b_ref, w_ref, hpn_ref, s_ref):
  xv = d_ref[...] * (p0_ref[...] + p1_ref[...] + hp_ref[...]) + b_ref[...]

  @pl.when(pl.program_id(0) == 0)
  def _():
    s_ref[...] = jnp.zeros_like(s_ref)

  s_ref[...] += jnp.sum(xv, axis=0, keepdims=True)
  hpn_ref[...] = (
      jnp.dot(xv, w_ref[...], preferred_element_type=jnp.float32) * d_ref[...]
  )


def _step(p0, p1, hp, d2, b2, w, rblk):
  n, dim = hp.shape
  grid = n // rblk
  return pl.pallas_call(
      _step_body,
      grid=(grid,),
      in_specs=[
          pl.BlockSpec((rblk, dim), lambda i: (i, 0)),
          pl.BlockSpec((rblk, dim), lambda i: (i, 0)),
          pl.BlockSpec((rblk, dim), lambda i: (i, 0)),
          pl.BlockSpec((rblk, 1), lambda i: (i, 0)),
          pl.BlockSpec((1, dim), lambda i: (0, 0)),
          pl.BlockSpec((dim, dim), lambda i: (0, 0)),
      ],
      out_specs=[
          pl.BlockSpec((rblk, dim), lambda i: (i, 0)),
          pl.BlockSpec((1, dim), lambda i: (0, 0)),
      ],
      out_shape=[
          jax.ShapeDtypeStruct((n, dim), jnp.float32),
          jax.ShapeDtypeStruct((1, dim), jnp.float32),
      ],
  )(p0, p1, hp, d2, b2, w)


def _colsum_body(p0_ref, p1_ref, hp_ref, d_ref, b_ref, s_ref):
  xv = d_ref[...] * (p0_ref[...] + p1_ref[...] + hp_ref[...]) + b_ref[...]

  @pl.when(pl.program_id(0) == 0)
  def _():
    s_ref[...] = jnp.zeros_like(s_ref)

  s_ref[...] += jnp.sum(xv, axis=0, keepdims=True)


def _colsum(p0, p1, hp, d2, b2, rblk):
  n, dim = hp.shape
  grid = n // rblk
  return pl.pallas_call(
      _colsum_body,
      grid=(grid,),
      in_specs=[
          pl.BlockSpec((rblk, dim), lambda i: (i, 0)),
          pl.BlockSpec((rblk, dim), lambda i: (i, 0)),
          pl.BlockSpec((rblk, dim), lambda i: (i, 0)),
          pl.BlockSpec((rblk, 1), lambda i: (i, 0)),
          pl.BlockSpec((1, dim), lambda i: (0, 0)),
      ],
      out_specs=pl.BlockSpec((1, dim), lambda i: (0, 0)),
      out_shape=jax.ShapeDtypeStruct((1, dim), jnp.float32),
  )(p0, p1, hp, d2, b2)


def kernel(x, edge_index, W1, b1, W2, b2, W3, b3, W4, b4):
  n, dim = x.shape
  e = edge_index.shape[1]
  assert dim % 16 == 0

  grain = NW * EBLK * NBUF
  ep = ((e + grain - 1) // grain) * grain
  npad = ep - e
  epw = ep // NW
  nblk = epw // EBLK
  src_flat = edge_index[0]
  dst_flat = edge_index[1]
  if npad:
    ar = jnp.arange(npad, dtype=jnp.int32)
    src_flat = jnp.concatenate([src_flat, (ar * 197) % n])
    dst_flat = jnp.concatenate([dst_flat, n + (ar % PADR)])
  src2 = src_flat.reshape(NW, epw)
  dst2 = dst_flat.reshape(NW, epw)

  deg_parts = _make_deg_kernel(ep, n)(dst2)
  deg = deg_parts[0, :n] + deg_parts[1, :n] + 1.0
  d2 = jnp.where(deg > 0, lax.rsqrt(deg), 0.0)[:, None]

  scatter = _make_scatter_kernel(ep, n, dim)
  rblk = 2000

  sums = []
  hp = _matscale(x, W1, d2, rblk)
  for b, wn in ((b1, W2), (b2, W3), (b3, W4)):
    parts = scatter(hp, src2, dst2)
    hp, s = _step(parts[0], parts[1], hp, d2, b.reshape(1, dim), wn, rblk)
    sums.append(s[0])
  parts = scatter(hp, src2, dst2)
  s = _colsum(parts[0], parts[1], hp, d2, b4.reshape(1, dim), rblk)
  sums.append(s[0])

  return jnp.concatenate(sums) / n


# speedup vs baseline: 1.0459x; 1.0188x over previous
import functools

import jax
import jax.numpy as jnp
from jax import lax
from jax.experimental import pallas as pl
from jax.experimental.pallas import tpu as pltpu
from jax.experimental.pallas import tpu_sc as plsc

NC = 2
NS = 16
NW = NC * NS
EBLK = 40
NBUF = 4
CHUNKS = tuple(range(0, EBLK - 15, 16)) + (() if EBLK % 16 == 0 else (EBLK - 16,))
PADR = 16


def _zero_vmem_1d(ref, n):
  zeros = jnp.zeros((16,), ref.dtype)

  def body(i):
    ref[pl.ds(i * 16, 16)] = zeros

  pl.loop(0, n // 16)(body)


def _make_deg_kernel(e, n):
  epw = e // NW
  nblk = epw // EBLK
  mesh = plsc.VectorSubcoreMesh(
      core_axis_name="c", subcore_axis_name="s", num_cores=NC, num_subcores=NS)

  nbuf = NBUF
  assert nblk % nbuf == 0

  @functools.partial(
      pl.kernel,
      mesh=mesh,
      out_type=jax.ShapeDtypeStruct((NC, n + PADR), jnp.float32),
      scratch_types=[
          pltpu.VMEM_SHARED((n + PADR,), jnp.float32),
          pltpu.VMEM((n + PADR,), jnp.float32),
          pltpu.VMEM((EBLK,), jnp.float32),
          pltpu.VMEM((epw,), jnp.int32),
      ]
      + [pltpu.VMEM((EBLK,), jnp.int32) for _ in range(nbuf)]
      + [pltpu.SemaphoreType.DMA for _ in range(nbuf + 1)],
  )
  def deg_kernel(dst_hbm, out_hbm, deg_sh, zbuf, ones_v, didx, *rest):
    dbufs = rest[:nbuf]
    ssems = rest[nbuf:2 * nbuf]
    isem = rest[2 * nbuf]
    cid = lax.axis_index("c")
    sid = lax.axis_index("s")
    wid = sid * NC + cid

    pltpu.async_copy(dst_hbm.at[wid], didx, isem)

    ones16 = jnp.ones((16,), jnp.float32)
    for o in CHUNKS:
      ones_v[pl.ds(o, 16)] = ones16

    @pl.when(sid == 0)
    def _():
      _zero_vmem_1d(zbuf, n + PADR)
      pltpu.sync_copy(zbuf, deg_sh)

    pltpu.make_async_copy(dst_hbm.at[wid], didx, isem).wait()
    plsc.subcore_barrier()

    def wait_scatter(k):
      pltpu.make_async_copy(ones_v, deg_sh.at[dbufs[k]], ssems[k]).wait()

    def body(i):
      for k in range(nbuf):
        j = i * nbuf + k

        @pl.when(i >= 1)
        def _():
          wait_scatter(k)

        base = j * EBLK
        for o in CHUNKS:
          dbufs[k][pl.ds(o, 16)] = didx[pl.ds(base + o, 16)]
        pltpu.async_copy(ones_v, deg_sh.at[dbufs[k]], ssems[k], add=True)

    pl.loop(0, nblk // nbuf)(body)

    for k in range(nbuf):
      wait_scatter(k)

    plsc.subcore_barrier()

    @pl.when(sid == 0)
    def _():
      pltpu.sync_copy(deg_sh, out_hbm.at[cid])

  return deg_kernel


def _make_scatter_kernel(e, n, d):
  epw = e // NW
  nblk = epw // EBLK
  mesh = plsc.VectorSubcoreMesh(
      core_axis_name="c", subcore_axis_name="s", num_cores=NC, num_subcores=NS)

  nbuf = NBUF
  zr = EBLK
  nch = n // zr
  bch = nch // NS
  xch = nch - bch * NS
  assert nblk % nbuf == 0 and nblk > 2 * nbuf and n % zr == 0 and zr % 8 == 0

  @functools.partial(
      pl.kernel,
      mesh=mesh,
      out_type=jax.ShapeDtypeStruct((NC, n, d), jnp.float32),
      scratch_types=[
          pltpu.VMEM_SHARED((n + PADR, d), jnp.float32),
          pltpu.VMEM((epw,), jnp.int32),
          pltpu.VMEM((epw,), jnp.int32),
      ]
      + [pltpu.VMEM((EBLK, d), jnp.float32) for _ in range(nbuf)]
      + [pltpu.VMEM((EBLK,), jnp.int32) for _ in range(nbuf)]
      + [pltpu.SemaphoreType.DMA for _ in range(2 * nbuf + 2)],
  )
  def scatter_kernel(hp_hbm, src_hbm, dst_hbm, out_hbm,
                     acc_sh, sidx, didx, *bufs_and_sems):
    bufs = bufs_and_sems[:nbuf]
    dbufs = bufs_and_sems[nbuf:2 * nbuf]
    gsems = bufs_and_sems[2 * nbuf:3 * nbuf]
    ssems = bufs_and_sems[3 * nbuf:4 * nbuf]
    zsem = bufs_and_sems[4 * nbuf]
    wsem = bufs_and_sems[4 * nbuf + 1]
    cid = lax.axis_index("c")
    sid = lax.axis_index("s")
    wid = sid * NC + cid
    ch0 = sid * bch + jnp.minimum(sid, xch)

    pltpu.async_copy(src_hbm.at[wid], sidx, gsems[0])
    pltpu.async_copy(dst_hbm.at[wid], didx, gsems[1])

    zeros16 = jnp.zeros((16,), jnp.float32)

    def zfill(i):
      def zrow(c):
        bufs[0][i, pl.ds(c * 16, 16)] = zeros16
      pl.loop(0, d // 16)(zrow)

    pl.loop(0, EBLK)(zfill)

    def zissue(i):
      pltpu.async_copy(bufs[0], acc_sh.at[pl.ds((ch0 + i) * zr, zr)], zsem)

    def zdrain(i):
      pltpu.make_async_copy(bufs[0], acc_sh.at[pl.ds(0, zr)], zsem).wait()

    pl.loop(0, bch)(zissue)

    @pl.when(sid < xch)
    def _():
      zissue(bch)

    pl.loop(0, bch)(zdrain)

    @pl.when(sid < xch)
    def _():
      zdrain(0)

    pltpu.make_async_copy(src_hbm.at[wid], sidx, gsems[0]).wait()
    pltpu.make_async_copy(dst_hbm.at[wid], didx, gsems[1]).wait()
    plsc.subcore_barrier()

    def issue_block(j, k):
      pltpu.async_copy(hp_hbm.at[sidx.at[pl.ds(j * EBLK, EBLK)]],
                       bufs[k], gsems[k])
      base = j * EBLK
      for o in CHUNKS:
        dbufs[k][pl.ds(o, 16)] = didx[pl.ds(base + o, 16)]

    def wait_block(k):
      pltpu.make_async_copy(hp_hbm.at[sidx.at[pl.ds(0, EBLK)]],
                            bufs[k], gsems[k]).wait()

    def issue_scatter(k):
      pltpu.async_copy(bufs[k], acc_sh.at[dbufs[k]], ssems[k], add=True)

    def wait_scatter(k):
      pltpu.make_async_copy(bufs[k], acc_sh.at[dbufs[k]], ssems[k]).wait()

    for b in range(nbuf - 1):
      issue_block(b, b)

    def body(i):
      for k in range(nbuf):
        j = i * nbuf + k
        kn = (k + nbuf - 1) % nbuf
        wait_block(k)
        issue_scatter(k)

        @pl.when(j + nbuf - 1 < nblk)
        def _():
          @pl.when(j >= 1)
          def _():
            wait_scatter(kn)

          issue_block(j + nbuf - 1, kn)

    pl.loop(0, nblk // nbuf)(body)

    for k in range(nbuf):
      wait_scatter(k)

    plsc.subcore_barrier()

    def wissue(i):
      row = (ch0 + i) * zr
      pltpu.async_copy(acc_sh.at[pl.ds(row, zr)],
                       out_hbm.at[cid, pl.ds(row, zr)], wsem)

    def wdrain(i):
      pltpu.make_async_copy(acc_sh.at[pl.ds(0, zr)],
                            out_hbm.at[cid, pl.ds(0, zr)], wsem).wait()

    pl.loop(0, bch)(wissue)

    @pl.when(sid < xch)
    def _():
      wissue(bch)

    pl.loop(0, bch)(wdrain)

    @pl.when(sid < xch)
    def _():
      wdrain(0)

  return scatter_kernel


def _matscale_body(x_ref, w_ref, d_ref, o_ref):
  o_ref[...] = (
      jnp.dot(x_ref[...], w_ref[...], preferred_element_type=jnp.float32)
      * d_ref[...]
  )


def _matscale(x, w, d2, rblk):
  n, dim = x.shape
  grid = n // rblk
  return pl.pallas_call(
      _matscale_body,
      grid=(grid,),
      in_specs=[
          pl.BlockSpec((rblk, dim), lambda i: (i, 0)),
          pl.BlockSpec((dim, dim), lambda i: (0, 0)),
          pl.BlockSpec((rblk, 1), lambda i: (i, 0)),
      ],
      out_specs=pl.BlockSpec((rblk, dim), lambda i: (i, 0)),
      out_shape=jax.ShapeDtypeStruct((n, dim), jnp.float32),
  )(x, w, d2)


def _step_body(p0_ref, p1_ref, hp_ref, d_ref, b_ref, w_ref, hpn_ref, s_ref):
  xv = d_ref[...] * (p0_ref[...] + p1_ref[...] + hp_ref[...]) + b_ref[...]

  @pl.when(pl.program_id(0) == 0)
  def _():
    s_ref[...] = jnp.zeros_like(s_ref)

  s_ref[...] += jnp.sum(xv, axis=0, keepdims=True)
  hpn_ref[...] = (
      jnp.dot(xv, w_ref[...], preferred_element_type=jnp.float32) * d_ref[...]
  )


def _step(p0, p1, hp, d2, b2, w, rblk):
  n, dim = hp.shape
  grid = n // rblk
  return pl.pallas_call(
      _step_body,
      grid=(grid,),
      in_specs=[
          pl.BlockSpec((rblk, dim), lambda i: (i, 0)),
          pl.BlockSpec((rblk, dim), lambda i: (i, 0)),
          pl.BlockSpec((rblk, dim), lambda i: (i, 0)),
          pl.BlockSpec((rblk, 1), lambda i: (i, 0)),
          pl.BlockSpec((1, dim), lambda i: (0, 0)),
          pl.BlockSpec((dim, dim), lambda i: (0, 0)),
      ],
      out_specs=[
          pl.BlockSpec((rblk, dim), lambda i: (i, 0)),
          pl.BlockSpec((1, dim), lambda i: (0, 0)),
      ],
      out_shape=[
          jax.ShapeDtypeStruct((n, dim), jnp.float32),
          jax.ShapeDtypeStruct((1, dim), jnp.float32),
      ],
  )(p0, p1, hp, d2, b2, w)


def _colsum_body(p0_ref, p1_ref, hp_ref, d_ref, b_ref, s_ref):
  xv = d_ref[...] * (p0_ref[...] + p1_ref[...] + hp_ref[...]) + b_ref[...]

  @pl.when(pl.program_id(0) == 0)
  def _():
    s_ref[...] = jnp.zeros_like(s_ref)

  s_ref[...] += jnp.sum(xv, axis=0, keepdims=True)


def _colsum(p0, p1, hp, d2, b2, rblk):
  n, dim = hp.shape
  grid = n // rblk
  return pl.pallas_call(
      _colsum_body,
      grid=(grid,),
      in_specs=[
          pl.BlockSpec((rblk, dim), lambda i: (i, 0)),
          pl.BlockSpec((rblk, dim), lambda i: (i, 0)),
          pl.BlockSpec((rblk, dim), lambda i: (i, 0)),
          pl.BlockSpec((rblk, 1), lambda i: (i, 0)),
          pl.BlockSpec((1, dim), lambda i: (0, 0)),
      ],
      out_specs=pl.BlockSpec((1, dim), lambda i: (0, 0)),
      out_shape=jax.ShapeDtypeStruct((1, dim), jnp.float32),
  )(p0, p1, hp, d2, b2)


def kernel(x, edge_index, W1, b1, W2, b2, W3, b3, W4, b4):
  n, dim = x.shape
  e = edge_index.shape[1]
  assert dim % 16 == 0

  grain = NW * EBLK * NBUF
  ep = ((e + grain - 1) // grain) * grain
  npad = ep - e
  epw = ep // NW
  nblk = epw // EBLK
  src_flat = edge_index[0]
  dst_flat = edge_index[1]
  if npad:
    ar = jnp.arange(npad, dtype=jnp.int32)
    src_flat = jnp.concatenate([src_flat, (ar * 197) % n])
    dst_flat = jnp.concatenate([dst_flat, n + (ar % PADR)])
  src2 = src_flat.reshape(NW, epw)
  dst2 = dst_flat.reshape(NW, epw)

  deg_parts = _make_deg_kernel(ep, n)(dst2)
  deg = deg_parts[0, :n] + deg_parts[1, :n] + 1.0
  d2 = jnp.where(deg > 0, lax.rsqrt(deg), 0.0)[:, None]

  scatter = _make_scatter_kernel(ep, n, dim)
  rblk = 2000

  sums = []
  hp = _matscale(x, W1, d2, rblk)
  for b, wn in ((b1, W2), (b2, W3), (b3, W4)):
    parts = scatter(hp, src2, dst2)
    hp, s = _step(parts[0], parts[1], hp, d2, b.reshape(1, dim), wn, rblk)
    sums.append(s[0])
  parts = scatter(hp, src2, dst2)
  s = _colsum(parts[0], parts[1], hp, d2, b4.reshape(1, dim), rblk)
  sums.append(s[0])

  return jnp.concatenate(sums) / n
